# baseline (speedup 1.0000x reference)
import functools

import jax
import jax.numpy as jnp
from jax import lax
from jax.experimental import pallas as pl
from jax.experimental.pallas import tpu as pltpu
from jax.experimental.pallas import tpu_sc as plsc

VOCAB = 100000
D = 128
NC, NS = 2, 16
NW = NC * NS
NBUF = 8
LP = 104


def _emb_kernel(B: int, S: int):
    bpw = B // NW
    n_steps = bpw // 2
    assert n_steps % NBUF == 0
    n_outer = n_steps // NBUF
    mesh = plsc.VectorSubcoreMesh(
        core_axis_name="c", subcore_axis_name="s", num_cores=NC, num_subcores=NS
    )

    @functools.partial(
        pl.kernel,
        out_type=jax.ShapeDtypeStruct((B, S, D), jnp.float32),
        mesh=mesh,
        scratch_types=[
            pltpu.VMEM((n_steps, LP), jnp.int32),
            pltpu.VMEM((NBUF * LP, D), jnp.float32),
            pltpu.SemaphoreType.DMA((NBUF,)),
            pltpu.SemaphoreType.DMA((NBUF,)),
        ],
    )
    def body(tbl_hbm, idx_hbm, out_hbm, idx_v, rows_v, gsem, wsem):
        wid = lax.axis_index("s") * NC + lax.axis_index("c")
        bat0 = wid * bpw
        pltpu.sync_copy(idx_hbm.at[wid], idx_v)

        def gather(j, slot):
            return pltpu.make_async_copy(
                tbl_hbm.at[idx_v.at[j]],
                rows_v.at[pl.ds(slot * LP, LP)],
                gsem.at[slot],
            )

        def writes(j, slot):
            b = bat0 + j * 2
            return (
                pltpu.make_async_copy(
                    rows_v.at[pl.ds(slot * LP, S)], out_hbm.at[b], wsem.at[slot]
                ),
                pltpu.make_async_copy(
                    rows_v.at[pl.ds(slot * LP + S, S)],
                    out_hbm.at[b + 1],
                    wsem.at[slot],
                ),
            )

        for s in range(NBUF):
            gather(s, s).start()

        def outer(jo, carry):
            j0 = jo * NBUF
            for s in range(NBUF):
                gather(j0 + s, s).wait()
                w0, w1 = writes(j0 + s, s)
                w0.start()
                w1.start()
            for s in range(NBUF):
                @pl.when(j0 + s + NBUF < n_steps)
                def _():
                    w0, w1 = writes(j0 + s, s)
                    w0.wait()
                    w1.wait()
                    gather(j0 + s + NBUF, s).start()
            return carry

        lax.fori_loop(0, n_outer, outer, 0)

        for s in range(NBUF):
            w0, w1 = writes(n_steps - NBUF + s, s)
            w0.wait()
            w1.wait()

    return body


def kernel(token_ids, weight):
    B, S = token_ids.shape
    bpw = B // NW
    n_steps = bpw // 2
    idx = token_ids.astype(jnp.int32).reshape(NW, n_steps, 2 * S)
    idx = jnp.pad(idx, ((0, 0), (0, 0), (0, LP - 2 * S)))
    return _emb_kernel(B, S)(weight, idx)


# speedup vs baseline: 3.5143x; 3.5143x over previous
import functools

import jax
import jax.numpy as jnp
from jax import lax
from jax.experimental import pallas as pl
from jax.experimental.pallas import tpu as pltpu
from jax.experimental.pallas import tpu_sc as plsc

VOCAB = 100000
D = 128
NC, NS = 2, 16
NW = NC * NS
NBUF = 2
GB = 8
LP = 104
NG = 4


def _emb_kernel(B: int, S: int):
    bpw = B // NW
    n_groups = bpw // GB
    n_steps = bpw // 2
    rows_per_group = GB * S
    assert n_groups % NBUF == 0
    mesh = plsc.VectorSubcoreMesh(
        core_axis_name="c", subcore_axis_name="s", num_cores=NC, num_subcores=NS
    )

    @functools.partial(
        pl.kernel,
        out_type=jax.ShapeDtypeStruct((B, S, D), jnp.float32),
        mesh=mesh,
        scratch_types=[
            pltpu.VMEM((n_steps, LP), jnp.int32),
            pltpu.VMEM((NBUF * rows_per_group, D), jnp.float32),
            pltpu.SemaphoreType.DMA((NBUF,)),
            pltpu.SemaphoreType.DMA((NBUF,)),
        ],
    )
    def body(tbl_hbm, idx_hbm, out_hbm, idx_v, rows_v, gsem, wsem):
        wid = lax.axis_index("s") * NC + lax.axis_index("c")
        bat0 = wid * bpw
        pltpu.sync_copy(idx_hbm.at[wid], idx_v)

        def gathers(g, slot):
            return [
                pltpu.make_async_copy(
                    tbl_hbm.at[idx_v.at[g * NG + k, pl.ds(0, 2 * S)]],
                    rows_v.at[pl.ds(slot * rows_per_group + k * 2 * S, 2 * S)],
                    gsem.at[slot],
                )
                for k in range(NG)
            ]

        def write(g, slot):
            src = rows_v.at[pl.ds(slot * rows_per_group, rows_per_group)]
            return pltpu.make_async_copy(
                src.reshape(GB, S, D),
                out_hbm.at[pl.ds(bat0 + g * GB, GB)],
                wsem.at[slot],
            )

        for s in range(NBUF):
            for cp in gathers(s, s):
                cp.start()

        def outer(go, carry):
            g0 = go * NBUF
            for s in range(NBUF):
                for cp in gathers(g0 + s, s):
                    cp.wait()
                write(g0 + s, s).start()
            for s in range(NBUF):
                @pl.when(g0 + s + NBUF < n_groups)
                def _():
                    write(g0 + s, s).wait()
                    for cp in gathers(g0 + s + NBUF, s):
                        cp.start()
            return carry

        lax.fori_loop(0, n_groups // NBUF, outer, 0)

        for s in range(NBUF):
            write(n_groups - NBUF + s, s).wait()

    return body


def kernel(token_ids, weight):
    B, S = token_ids.shape
    bpw = B // NW
    n_steps = bpw // 2
    idx = token_ids.astype(jnp.int32).reshape(NW, n_steps, 2 * S)
    idx = jnp.pad(idx, ((0, 0), (0, 0), (0, LP - 2 * S)))
    return _emb_kernel(B, S)(weight, idx)
